# baseline (speedup 1.0000x reference)
import functools

import jax
import jax.numpy as jnp
from jax import lax
from jax.experimental import pallas as pl
from jax.experimental.pallas import tpu as pltpu
from jax.experimental.pallas import tpu_sc as plsc

_STATES = 1000000
_ACTIONS = 16
_BATCH = 16384
_LANES = 128
_PACK = _LANES // _ACTIONS


@functools.cache
def _build_gather():
    info = plsc.get_sparse_core_info()
    num_cores, num_subcores = info.num_cores, info.num_subcores
    num_workers = num_cores * num_subcores
    b_per_w = _BATCH // num_workers
    n_blk = b_per_w // 16
    out_wide_rows = _BATCH * _ACTIONS // _LANES
    mesh = plsc.VectorSubcoreMesh(core_axis_name="c", subcore_axis_name="s")

    @functools.partial(
        pl.kernel,
        mesh=mesh,
        out_type=jax.ShapeDtypeStruct((out_wide_rows, _LANES), jnp.float32),
        compiler_params=pltpu.CompilerParams(needs_layout_passes=False),
        scratch_types=[
            pltpu.VMEM((b_per_w,), jnp.int32),
            pltpu.VMEM((b_per_w,), jnp.int32),
            pltpu.VMEM((b_per_w, _LANES), jnp.float32),
            pltpu.VMEM((b_per_w // _PACK, _LANES), jnp.float32),
            pltpu.SemaphoreType.DMA,
        ],
    )
    def gather_kernel(values_hbm, idx_hbm, out_hbm, idx_v, row_v, wide_v,
                      out_v, sem):
        wid = lax.axis_index("s") * num_cores + lax.axis_index("c")
        base = wid * b_per_w
        pltpu.sync_copy(idx_hbm.at[pl.ds(base, b_per_w)], idx_v)

        def row_body(k, _):
            v = idx_v[pl.ds(k * 16, 16)]
            row_v[pl.ds(k * 16, 16)] = lax.shift_right_logical(v, 3)
            return _

        lax.fori_loop(0, n_blk, row_body, None, unroll=4)

        pltpu.async_copy(values_hbm.at[row_v], wide_v, sem).wait()

        lane = lax.iota(jnp.int32, 16)

        def ext_body(k, _):
            b = k * 16 + lane
            idxs = idx_v[pl.ds(k * 16, 16)]
            src_col = (idxs & 7) * 16
            dst_row = lax.shift_right_logical(b, 3)
            dst_col = (b & 7) * 16
            for j in range(_ACTIONS):
                vals = plsc.load_gather(wide_v, [b, src_col + j])
                plsc.store_scatter(out_v, [dst_row, dst_col + j], vals)
            return _

        lax.fori_loop(0, n_blk, ext_body, None)

        pltpu.sync_copy(
            out_v, out_hbm.at[pl.ds(wid * (b_per_w // _PACK), b_per_w // _PACK)]
        )

    return gather_kernel


def kernel(state, values):
    idx = state.astype(jnp.int32)
    wide = values.reshape(_STATES * _ACTIONS // _LANES, _LANES)
    out = _build_gather()(wide, idx)
    return out.reshape(_BATCH, _ACTIONS)
